# Initial kernel scaffold; baseline (speedup 1.0000x reference)
import jax
import jax.numpy as jnp
from jax.experimental import pallas as pl


def kernel(x, past_length, pos_table):
    raise NotImplementedError("write your pallas kernel here")


# speedup vs baseline: 1.1073x; 1.1073x over previous
import functools

import jax
import jax.numpy as jnp
from jax import lax
from jax.experimental import pallas as pl
from jax.experimental.pallas import tpu as pltpu
from jax.experimental.pallas import tpu_sc as plsc

_B, _S, _D = 4, 8192, 768
_NC, _NS = 2, 16
_NW = _NC * _NS
_RPW = _S // _NW
_C = 16
_NCH = _RPW // _C
_L = 16
_JV = _D // _L

_mesh = plsc.VectorSubcoreMesh(core_axis_name="c", subcore_axis_name="s")


@functools.partial(
    pl.kernel,
    mesh=_mesh,
    out_type=jax.ShapeDtypeStruct((_B * _S, _D), jnp.float32),
    scratch_types=[
        pltpu.VMEM((_C, _D), jnp.float32),
        pltpu.VMEM((_C, _D), jnp.float32),
        pltpu.VMEM((_C, _D), jnp.float32),
        pltpu.VMEM((_C, _D), jnp.float32),
        pltpu.VMEM((_C, _D), jnp.float32),
        pltpu.VMEM((_C, _D), jnp.float32),
        pltpu.VMEM((_L,), jnp.int32),
        pltpu.SemaphoreType.DMA,
        pltpu.SemaphoreType.DMA,
        pltpu.SemaphoreType.DMA,
        pltpu.SemaphoreType.DMA,
        pltpu.SemaphoreType.DMA,
        pltpu.SemaphoreType.DMA,
        pltpu.SemaphoreType.DMA,
        pltpu.SemaphoreType.DMA,
        pltpu.SemaphoreType.DMA,
        pltpu.SemaphoreType.DMA,
    ],
)
def _pos_add(x_hbm, plen_hbm, pos_hbm, out_hbm,
             pos0, pos1, xb0, xb1, xb2, xb3, plen_v,
             psem0, psem1, isem0, isem1, isem2, isem3,
             osem0, osem1, osem2, osem3):
    pos_v = (pos0, pos1)
    x_v = (xb0, xb1, xb2, xb3)
    psem = (psem0, psem1)
    isem = (isem0, isem1, isem2, isem3)
    osem = (osem0, osem1, osem2, osem3)

    wid = lax.axis_index("s") * _NC + lax.axis_index("c")
    s0 = wid * _RPW
    pltpu.sync_copy(plen_hbm, plen_v)
    past = plen_v[...]

    def pos_idx(k):
        return jnp.clip(s0 + k * _C + past + lax.iota(jnp.int32, _L),
                        0, _S - 1)

    def pos_start(k, pb):
        pltpu.async_copy(pos_hbm.at[pos_idx(k)], pos_v[pb], psem[pb])

    def pos_wait(k, pb):
        pltpu.make_async_copy(pos_hbm.at[pos_idx(k)], pos_v[pb],
                              psem[pb]).wait()

    def x_base(k, b):
        return b * _S + s0 + k * _C

    def x_start(k, b):
        pltpu.async_copy(x_hbm.at[pl.ds(x_base(k, b), _C)], x_v[b],
                         isem[b])

    def x_drain(b):
        pltpu.make_async_copy(x_hbm.at[pl.ds(0, _C)], x_v[b],
                              isem[b]).wait()

    def out_start(k, b):
        pltpu.async_copy(x_v[b], out_hbm.at[pl.ds(x_base(k, b), _C)],
                         osem[b])

    def out_drain(b):
        pltpu.make_async_copy(x_v[b], out_hbm.at[pl.ds(0, _C)],
                              osem[b]).wait()

    pos_start(0, 0)
    x_start(0, 0)
    x_start(0, 1)

    def outer(i, carry):
        for j in range(2):
            k = 2 * i + j
            pp = j
            for b in range(_B):
                x_drain(b)
                if b == 0:
                    pos_wait(k, pp)
                    if j == 0:
                        pos_start(k + 1, 1 - pp)
                    else:
                        @pl.when(i < _NCH // 2 - 1)
                        def _(k=k, pp=pp):
                            pos_start(k + 1, 1 - pp)
                nb = (b + 2) % _B
                nk = k if b < 2 else k + 1
                if j == 0 and b < 2:
                    @pl.when(i > 0)
                    def _(nb=nb):
                        out_drain(nb)
                    x_start(nk, nb)
                elif j == 1 and b >= 2:
                    @pl.when(i < _NCH // 2 - 1)
                    def _(nk=nk, nb=nb):
                        out_drain(nb)
                        x_start(nk, nb)
                else:
                    out_drain(nb)
                    x_start(nk, nb)
                for r in range(_C):
                    @plsc.parallel_loop(0, _JV, unroll=8)
                    def _vec(jj, r=r, b=b, pp=pp):
                        sl = pl.ds(jj * _L, _L)
                        plsc.addupdate(x_v[b].at[r, sl], pos_v[pp][r, sl])
                out_start(k, b)
        return carry

    lax.fori_loop(0, _NCH // 2, outer, 0)
    for b in range(_B):
        out_drain(b)


def kernel(x, past_length, pos_table):
    plen = jnp.full((_L,), past_length, dtype=jnp.int32)
    out = _pos_add(x.reshape(_B * _S, _D), plen, pos_table)
    return out.reshape(_B, _S, _D)
